# baseline (speedup 1.0000x reference)
import functools

import jax
import jax.numpy as jnp
from jax.experimental import pallas as pl
from jax.experimental.pallas import tpu as pltpu

N = 10000
F = 128
H = 256
A = 64
NH = A // 2
E = 160000


def _mean_agg(x, ei):
    src, dst = ei[0], ei[1]
    msgs = x[src]
    s = jax.ops.segment_sum(msgs, dst, num_segments=N)
    c = jax.ops.segment_sum(jnp.ones((E,), jnp.float32), dst, num_segments=N)
    return s / jnp.clip(c, 1.0)[:, None]


def _max_agg(x, ei):
    src, dst = ei[0], ei[1]
    agg = jax.ops.segment_max(x[src], dst, num_segments=N)
    return jnp.where(jnp.isfinite(agg), agg, 0.0)


def _tail_kernel(x_ref, w1_ref, b1_ref, w2_ref, b2_ref, wo_ref, bo_ref, o_ref):
    x = x_ref[...]
    h1 = jnp.dot(x, w1_ref[...], preferred_element_type=jnp.float32) + b1_ref[...]
    h2 = jnp.dot(h1, w2_ref[...], preferred_element_type=jnp.float32) + b2_ref[...]
    o_ref[...] = jnp.dot(h2, wo_ref[...], preferred_element_type=jnp.float32) + bo_ref[...]


def _tail(x, Wlin1, blin1, Wlin2, blin2, Wo, bo):
    blk = 1000
    grid = (N // blk,)
    return pl.pallas_call(
        _tail_kernel,
        grid=grid,
        in_specs=[
            pl.BlockSpec((blk, H), lambda i: (i, 0)),
            pl.BlockSpec((H, H), lambda i: (0, 0)),
            pl.BlockSpec((H,), lambda i: (0,)),
            pl.BlockSpec((H, H), lambda i: (0, 0)),
            pl.BlockSpec((H,), lambda i: (0,)),
            pl.BlockSpec((H, A), lambda i: (0, 0)),
            pl.BlockSpec((A,), lambda i: (0,)),
        ],
        out_specs=pl.BlockSpec((blk, A), lambda i: (i, 0)),
        out_shape=jax.ShapeDtypeStruct((N, A), jnp.float32),
    )(x, Wlin1, blin1, Wlin2, blin2, Wo, bo)


def kernel(actions, obs, eic, eid, eit, W1l, W1r, b1, W2l, W2r, b2, W3l, W3r,
           b3, W4l, W4r, b4, W5l, W5r, b5, Wlin1, blin1, Wlin2, blin2, Wo, bo):
    x = _mean_agg(obs, eic) @ W1l + b1 + obs @ W1r
    x = _mean_agg(x, eit) @ W2l + b2 + x @ W2r
    x = _max_agg(x, eic) @ W3l + b3 + x @ W3r
    x = _mean_agg(x, eid) @ W4l + b4 + x @ W4r
    x = _mean_agg(x, eic) @ W5l + b5 + x @ W5r
    x = _tail(x, Wlin1, blin1, Wlin2, blin2, Wo, bo)
    a = actions.reshape(-1, 2)
    starts = x[:, :NH][a[:, 0]]
    dests = x[:, NH:][a[:, 1]]
    probs = jax.nn.softmax(jnp.sum(starts * dests, axis=-1))
    return probs[None, :]
